# Initial kernel scaffold; baseline (speedup 1.0000x reference)
import jax
import jax.numpy as jnp
from jax.experimental import pallas as pl


def kernel(units, keys):
    raise NotImplementedError("write your pallas kernel here")


# speedup vs baseline: 3.3660x; 3.3660x over previous
import jax
import jax.numpy as jnp
from jax.experimental import pallas as pl
from jax.experimental.pallas import tpu as pltpu
from jax.experimental.pallas import tpu_sc as plsc

Q = 1024
D = 128
K = 100000
BK = 2000
NBLK = K // BK
INDEX_RATE_ = 0.5

GATHER_WINDOW = 128


def _argmin_body(u_ref, kb_ref, idx_ref, best_ref, bidx_ref):
    k = pl.program_id(0)

    @pl.when(k == 0)
    def _init():
        best_ref[...] = jnp.full_like(best_ref, jnp.inf)
        bidx_ref[...] = jnp.zeros_like(bidx_ref)

    u = u_ref[...]
    kb = kb_ref[...]
    sq_u = jnp.sum(u * u, axis=1, keepdims=True)
    sq_k = jnp.sum(kb * kb, axis=1, keepdims=True).T
    dots = jax.lax.dot_general(
        u, kb,
        dimension_numbers=(((1,), (1,)), ((), ())),
        preferred_element_type=jnp.float32,
    )
    dist = sq_u - 2.0 * dots + sq_k

    bmin = jnp.min(dist, axis=1, keepdims=True)
    barg = jnp.argmin(dist, axis=1).astype(jnp.int32)
    gidx = k * BK + barg[:, None]

    better = bmin < best_ref[...]
    bidx_ref[...] = jnp.where(better, gidx, bidx_ref[...])
    best_ref[...] = jnp.where(better, bmin, best_ref[...])

    @pl.when(k == NBLK - 1)
    def _done():
        idx_ref[...] = bidx_ref[...]


def _top1_indices(units, keys):
    return pl.pallas_call(
        _argmin_body,
        grid=(NBLK,),
        in_specs=[
            pl.BlockSpec((Q, D), lambda k: (0, 0)),
            pl.BlockSpec((BK, D), lambda k: (k, 0)),
        ],
        out_specs=pl.BlockSpec((Q, 1), lambda k: (0, 0)),
        out_shape=jax.ShapeDtypeStruct((Q, 1), jnp.int32),
        scratch_shapes=[
            pltpu.VMEM((Q, 1), jnp.float32),
            pltpu.VMEM((Q, 1), jnp.int32),
        ],
    )(units, keys)


def _gather_blend(units, keys, idx_row):
    mesh = plsc.VectorSubcoreMesh(core_axis_name="c", subcore_axis_name="s")

    @pl.kernel(out_type=jax.ShapeDtypeStruct((Q, D), jnp.float32), mesh=mesh)
    def sc_kernel(keys_hbm, idx_hbm, units_hbm, o_hbm):
        def body(i_vmem, u_vmem, o_vmem):
            pltpu.sync_copy(keys_hbm.at[i_vmem.at[0]], o_vmem)

            @pl.loop(0, GATHER_WINDOW)
            def _(r):
                @pl.loop(0, D, step=16)
                def _(c):
                    slc = (pl.ds(r, 1), pl.ds(c, 16))
                    o_vmem.at[*slc][...] = (
                        (1.0 - INDEX_RATE_) * u_vmem.at[*slc][...]
                        + INDEX_RATE_ * o_vmem.at[*slc][...]
                    )

        pltpu.emit_pipeline(
            body,
            grid=(Q // GATHER_WINDOW,),
            in_specs=[
                pl.BlockSpec((1, GATHER_WINDOW), lambda i: (0, i)),
                pl.BlockSpec((GATHER_WINDOW, D), lambda i: (i, 0)),
            ],
            out_specs=[pl.BlockSpec((GATHER_WINDOW, D), lambda i: (i, 0))],
            core_axis_name=("c", "s"),
            dimension_semantics=(pltpu.PARALLEL,),
        )(idx_hbm, units_hbm, o_hbm)

    return sc_kernel(keys, idx_row, units)


def kernel(units, keys):
    idx = _top1_indices(units, keys)
    idx_row = idx.reshape(1, Q)
    return _gather_blend(units, keys, idx_row)


# speedup vs baseline: 4.0207x; 1.1945x over previous
import jax
import jax.numpy as jnp
from jax.experimental import pallas as pl
from jax.experimental.pallas import tpu as pltpu
from jax.experimental.pallas import tpu_sc as plsc

Q = 1024
D = 128
K = 100000
BK = 2048
CHUNK = 128
NCH = BK // CHUNK
NBLK = (K + BK - 1) // BK
LAST = NBLK - 1
_LAST_FULL = (K - LAST * BK) // CHUNK
_TAIL_LANES = K - LAST * BK - _LAST_FULL * CHUNK
INDEX_RATE_ = 0.5

GATHER_WINDOW = 128


def _argmin_body(u_ref, kb_ref, idx_ref, squ_ref, rmin_ref, rarg_ref):
    k = pl.program_id(0)

    @pl.when(k == 0)
    def _init():
        u = u_ref[...]
        squ_ref[...] = jnp.broadcast_to(
            jnp.sum(u * u, axis=1, keepdims=True), (Q, CHUNK))
        rmin_ref[...] = jnp.full_like(rmin_ref, jnp.inf)
        rarg_ref[...] = jnp.zeros_like(rarg_ref)

    kb = kb_ref[...]
    dots = jax.lax.dot_general(
        u_ref[...], kb,
        dimension_numbers=(((1,), (1,)), ((), ())),
        preferred_element_type=jnp.float32,
    )
    sqk = jnp.sum(kb * kb, axis=1, keepdims=True).T
    squ = squ_ref[...]

    def chunk_update(c, tail_lanes=None):
        dc = squ - 2.0 * dots[:, c * CHUNK:(c + 1) * CHUNK] \
            + sqk[:, c * CHUNK:(c + 1) * CHUNK]
        if tail_lanes is not None:
            lane = jax.lax.broadcasted_iota(jnp.int32, (Q, CHUNK), 1)
            dc = jnp.where(lane < tail_lanes, dc, jnp.inf)
        m = k * NCH + c
        cmp = dc < rmin_ref[...]
        rmin_ref[...] = jnp.where(cmp, dc, rmin_ref[...])
        rarg_ref[...] = jnp.where(cmp, m, rarg_ref[...])

    @pl.when(k < LAST)
    def _main():
        for c in range(NCH):
            chunk_update(c)

    @pl.when(k == LAST)
    def _last():
        for c in range(_LAST_FULL):
            chunk_update(c)
        chunk_update(_LAST_FULL, tail_lanes=_TAIL_LANES)
        v = rmin_ref[...]
        lane = jax.lax.broadcasted_iota(jnp.int32, (Q, CHUNK), 1)
        col = rarg_ref[...] * CHUNK + lane
        vm = jnp.min(v, axis=1, keepdims=True)
        colm = jnp.where(v == vm, col, jnp.int32(2**30))
        idx_ref[...] = jnp.min(colm, axis=1, keepdims=True)


def _top1_indices(units, keys):
    return pl.pallas_call(
        _argmin_body,
        grid=(NBLK,),
        in_specs=[
            pl.BlockSpec((Q, D), lambda k: (0, 0)),
            pl.BlockSpec((BK, D), lambda k: (k, 0)),
        ],
        out_specs=pl.BlockSpec((Q, 1), lambda k: (0, 0)),
        out_shape=jax.ShapeDtypeStruct((Q, 1), jnp.int32),
        scratch_shapes=[
            pltpu.VMEM((Q, CHUNK), jnp.float32),
            pltpu.VMEM((Q, CHUNK), jnp.float32),
            pltpu.VMEM((Q, CHUNK), jnp.int32),
        ],
    )(units, keys)


def _gather_blend(units, keys, idx_row):
    mesh = plsc.VectorSubcoreMesh(core_axis_name="c", subcore_axis_name="s")

    @pl.kernel(out_type=jax.ShapeDtypeStruct((Q, D), jnp.float32), mesh=mesh)
    def sc_kernel(keys_hbm, idx_hbm, units_hbm, o_hbm):
        def body(i_vmem, u_vmem, o_vmem):
            pltpu.sync_copy(keys_hbm.at[i_vmem.at[0]], o_vmem)

            @pl.loop(0, GATHER_WINDOW)
            def _(r):
                @pl.loop(0, D, step=16)
                def _(c):
                    slc = (pl.ds(r, 1), pl.ds(c, 16))
                    o_vmem.at[*slc][...] = (
                        (1.0 - INDEX_RATE_) * u_vmem.at[*slc][...]
                        + INDEX_RATE_ * o_vmem.at[*slc][...]
                    )

        pltpu.emit_pipeline(
            body,
            grid=(Q // GATHER_WINDOW,),
            in_specs=[
                pl.BlockSpec((1, GATHER_WINDOW), lambda i: (0, i)),
                pl.BlockSpec((GATHER_WINDOW, D), lambda i: (i, 0)),
            ],
            out_specs=[pl.BlockSpec((GATHER_WINDOW, D), lambda i: (i, 0))],
            core_axis_name=("c", "s"),
            dimension_semantics=(pltpu.PARALLEL,),
        )(idx_hbm, units_hbm, o_hbm)

    return sc_kernel(keys, idx_row, units)


def kernel(units, keys):
    idx = _top1_indices(units, keys)
    idx_row = idx.reshape(1, Q)
    return _gather_blend(units, keys, idx_row)


# speedup vs baseline: 5.9180x; 1.4719x over previous
import jax
import jax.numpy as jnp
from jax.experimental import pallas as pl
from jax.experimental.pallas import tpu as pltpu
from jax.experimental.pallas import tpu_sc as plsc

Q = 1024
D = 128
K = 100000
BK = 4096
CHUNK = 128
NCH = BK // CHUNK
NBLK = (K + BK - 1) // BK
LAST = NBLK - 1
HALF = (NBLK + 1) // 2
_LAST_FULL = (K - LAST * BK) // CHUNK
_TAIL_LANES = K - LAST * BK - _LAST_FULL * CHUNK
INDEX_RATE_ = 0.5

GATHER_WINDOW = 128

MM = 256


def _argmin_body(u_ref, kb_ref, val_ref, idx_ref,
                 u2_ref, squ_ref, rmin_ref, rarg_ref):
    i = pl.program_id(0)
    j = pl.program_id(1)
    b = i * HALF + j

    @pl.when(j == 0)
    def _init():
        u = u_ref[...]
        u2_ref[...] = -2.0 * u
        squ_ref[...] = jnp.broadcast_to(
            jnp.sum(u * u, axis=1, keepdims=True), (Q, CHUNK))
        rmin_ref[...] = jnp.full_like(rmin_ref, jnp.inf)
        rarg_ref[...] = jnp.zeros_like(rarg_ref)

    kb = kb_ref[...]
    sqk = jnp.sum(kb * kb, axis=1, keepdims=True).T
    squ = squ_ref[...]
    u2 = u2_ref[...]

    def mm_chunk(c2):
        return jax.lax.dot_general(
            u2, kb[c2 * MM:(c2 + 1) * MM, :],
            dimension_numbers=(((1,), (1,)), ((), ())),
            preferred_element_type=jnp.float32,
        )

    def chunk_update(dots2, c, tail_lanes=None):
        lo = (c % 2) * CHUNK
        dc = (squ + dots2[:, lo:lo + CHUNK]) \
            + sqk[:, c * CHUNK:(c + 1) * CHUNK]
        if tail_lanes is not None:
            lane = jax.lax.broadcasted_iota(jnp.int32, (Q, CHUNK), 1)
            dc = jnp.where(lane < tail_lanes, dc, jnp.inf)
        m = b * NCH + c
        cmp = dc < rmin_ref[...]
        rmin_ref[...] = jnp.where(cmp, dc, rmin_ref[...])
        rarg_ref[...] = jnp.where(cmp, m, rarg_ref[...])

    @pl.when(b < LAST)
    def _main():
        for c2 in range(NCH // 2):
            dots2 = mm_chunk(c2)
            chunk_update(dots2, 2 * c2)
            chunk_update(dots2, 2 * c2 + 1)

    @pl.when(b == LAST)
    def _last():
        for c2 in range(_LAST_FULL // 2):
            dots2 = mm_chunk(c2)
            chunk_update(dots2, 2 * c2)
            chunk_update(dots2, 2 * c2 + 1)
        dots2 = mm_chunk(_LAST_FULL // 2)
        chunk_update(dots2, _LAST_FULL - 1)
        chunk_update(dots2, _LAST_FULL, tail_lanes=_TAIL_LANES)

    @pl.when(j == HALF - 1)
    def _extract():
        v = rmin_ref[...]
        lane = jax.lax.broadcasted_iota(jnp.int32, (Q, CHUNK), 1)
        col = rarg_ref[...] * CHUNK + lane
        vm = jnp.min(v, axis=1, keepdims=True)
        colm = jnp.where(v == vm, col, jnp.int32(2**30))
        cm = jnp.min(colm, axis=1, keepdims=True)
        val_ref[...] = vm.T.reshape(1, 1, Q)
        idx_ref[...] = cm.T.reshape(1, 1, Q)


def _top1_candidates(units, keys):
    return pl.pallas_call(
        _argmin_body,
        grid=(2, HALF),
        in_specs=[
            pl.BlockSpec((Q, D), lambda i, j: (0, 0)),
            pl.BlockSpec((BK, D),
                         lambda i, j: (jnp.minimum(i * HALF + j, LAST), 0)),
        ],
        out_specs=[
            pl.BlockSpec((1, 1, Q), lambda i, j: (i, 0, 0)),
            pl.BlockSpec((1, 1, Q), lambda i, j: (i, 0, 0)),
        ],
        out_shape=[
            jax.ShapeDtypeStruct((2, 1, Q), jnp.float32),
            jax.ShapeDtypeStruct((2, 1, Q), jnp.int32),
        ],
        scratch_shapes=[
            pltpu.VMEM((Q, D), jnp.float32),
            pltpu.VMEM((Q, CHUNK), jnp.float32),
            pltpu.VMEM((Q, CHUNK), jnp.float32),
            pltpu.VMEM((Q, CHUNK), jnp.int32),
        ],
        compiler_params=pltpu.CompilerParams(
            dimension_semantics=("parallel", "arbitrary"),
        ),
    )(units, keys)


def _merge_gather_blend(units, keys, vals, idxs):
    mesh = plsc.VectorSubcoreMesh(core_axis_name="c", subcore_axis_name="s")
    W = GATHER_WINDOW

    @pl.kernel(out_type=jax.ShapeDtypeStruct((Q, D), jnp.float32), mesh=mesh)
    def sc_kernel(keys_hbm, vals_hbm, idxs_hbm, units_hbm, o_hbm):
        def body(v_vmem, i_vmem, u_vmem, o_vmem):
            @pl.loop(0, W, step=16)
            def _(c):
                v0 = v_vmem.at[pl.ds(0, 1), pl.ds(c, 16)][...]
                v1 = v_vmem.at[pl.ds(1, 1), pl.ds(c, 16)][...]
                i0 = i_vmem.at[pl.ds(0, 1), pl.ds(c, 16)][...]
                i1 = i_vmem.at[pl.ds(1, 1), pl.ds(c, 16)][...]
                i_vmem.at[pl.ds(0, 1), pl.ds(c, 16)][...] = \
                    jnp.where(v1 < v0, i1, i0)

            pltpu.sync_copy(keys_hbm.at[i_vmem.at[0]], o_vmem)

            @pl.loop(0, W)
            def _(r):
                @pl.loop(0, D, step=16)
                def _(c):
                    slc = (pl.ds(r, 1), pl.ds(c, 16))
                    o_vmem.at[*slc][...] = (
                        (1.0 - INDEX_RATE_) * u_vmem.at[*slc][...]
                        + INDEX_RATE_ * o_vmem.at[*slc][...]
                    )

        pltpu.emit_pipeline(
            body,
            grid=(Q // W,),
            in_specs=[
                pl.BlockSpec((2, W), lambda i: (0, i)),
                pl.BlockSpec((2, W), lambda i: (0, i)),
                pl.BlockSpec((W, D), lambda i: (i, 0)),
            ],
            out_specs=[pl.BlockSpec((W, D), lambda i: (i, 0))],
            core_axis_name=("c", "s"),
            dimension_semantics=(pltpu.PARALLEL,),
        )(vals_hbm, idxs_hbm, units_hbm, o_hbm)

    return sc_kernel(keys, vals, idxs, units)


def kernel(units, keys):
    vals, idxs = _top1_candidates(units, keys)
    return _merge_gather_blend(units, keys,
                               vals.reshape(2, Q), idxs.reshape(2, Q))
